# baseline (speedup 1.0000x reference)
import functools

import jax
import jax.numpy as jnp
from jax import lax
from jax.experimental import pallas as pl
from jax.experimental.pallas import tpu as pltpu
from jax.experimental.pallas import tpu_sc as plsc

NUM_CORES = 2
NUM_SUBCORES = 16
NUM_WORKERS = NUM_CORES * NUM_SUBCORES
LANES = 16
BATCH = 16384
EMB = 64
BPW = BATCH // NUM_WORKERS
CHUNK = 128
NCHUNK = BPW // CHUNK

_GATHER_DNUMS = lax.GatherDimensionNumbers(
    offset_dims=(), collapsed_slice_dims=(0,), start_index_map=(0,))


def _shuffle(x, perm):
    return lax.gather(x, perm[:, None], dimension_numbers=_GATHER_DNUMS,
                      slice_sizes=(1,),
                      mode=lax.GatherScatterMode.PROMISE_IN_BOUNDS)


def _body(u_hbm, v_hbm, ue_hbm, ve_hbm, out_hbm,
          u_idx, v_idx, ue_v, ve_v, out_v, sem):
    wid = lax.axis_index("s") * NUM_CORES + lax.axis_index("c")
    base = wid * BPW

    for j in range(NCHUNK):
        pltpu.sync_copy(u_hbm.at[pl.ds(base + j * CHUNK, CHUNK)], u_idx.at[j])
        pltpu.sync_copy(v_hbm.at[pl.ds(base + j * CHUNK, CHUNK)], v_idx.at[j])

    for j in range(NCHUNK):
        pltpu.async_copy(ue_hbm.at[u_idx.at[j]], ue_v.at[pl.ds(j * CHUNK, CHUNK)], sem)
        pltpu.async_copy(ve_hbm.at[v_idx.at[j]], ve_v.at[pl.ds(j * CHUNK, CHUNK)], sem)
    for j in range(NCHUNK):
        pltpu.make_async_copy(ue_hbm.at[u_idx.at[j]], ue_v.at[pl.ds(j * CHUNK, CHUNK)], sem).wait()
        pltpu.make_async_copy(ve_hbm.at[v_idx.at[j]], ve_v.at[pl.ds(j * CHUNK, CHUNK)], sem).wait()

    lanes = lax.iota(jnp.int32, LANES)
    perms = [lanes ^ (1 << t) for t in range(4)]

    def group(g, carry):
        gbase = pl.multiple_of(g * LANES, LANES)
        sums = jnp.zeros((LANES,), jnp.float32)
        for r in range(LANES):
            row = gbase + r
            acc = ue_v[row, pl.ds(0, LANES)] * ve_v[row, pl.ds(0, LANES)]
            for c in range(1, EMB // LANES):
                acc = acc + (ue_v[row, pl.ds(c * LANES, LANES)]
                             * ve_v[row, pl.ds(c * LANES, LANES)])
            for t in range(4):
                acc = acc + _shuffle(acc, perms[t])
            sums = jnp.where(lanes == r, acc, sums)
        out_v[pl.ds(gbase, LANES)] = sums
        return carry

    lax.fori_loop(0, BPW // LANES, group, 0)

    pltpu.sync_copy(out_v, out_hbm.at[pl.ds(base, BPW)])


@jax.jit
def kernel(u, v, user_emb, item_emb):
    mesh = plsc.VectorSubcoreMesh(core_axis_name="c", subcore_axis_name="s",
                                  num_cores=NUM_CORES, num_subcores=NUM_SUBCORES)
    run = pl.kernel(
        _body,
        out_type=jax.ShapeDtypeStruct((BATCH,), jnp.float32),
        mesh=mesh,
        scratch_types=[
            pltpu.VMEM((NCHUNK, CHUNK), jnp.int32),
            pltpu.VMEM((NCHUNK, CHUNK), jnp.int32),
            pltpu.VMEM((BPW, EMB), jnp.float32),
            pltpu.VMEM((BPW, EMB), jnp.float32),
            pltpu.VMEM((BPW,), jnp.float32),
            pltpu.SemaphoreType.DMA,
        ],
        compiler_params=pltpu.CompilerParams(use_tc_tiling_on_sc=False),
    )
    return run(u, v, user_emb, item_emb)
